# baseline (speedup 1.0000x reference)
import functools

import numpy as _NP

import jax
import jax.numpy as jnp
from jax import lax
from jax.experimental import pallas as pl
from jax.experimental.pallas import tpu as pltpu
from jax.experimental.pallas import tpu_sc as plsc

_T, _D, _E, _F, _K = 2048, 1024, 8, 512, 2
_N = _T * _K
_B = 128
_NT = 40
_NPAD = _NT * _B
_NTE = 48
_NC, _NS, _L = 2, 16, 16


def _router_body(x_ref, gw_ref, w_ref, eid_ref, hist_ref):
    x = x_ref[...]
    logits = jnp.dot(x, gw_ref[...], preferred_element_type=jnp.float32)
    p = jax.nn.softmax(logits, axis=-1)
    col = lax.broadcasted_iota(jnp.int32, (_T, _E), 1)
    rank = jnp.zeros((_T, _E), jnp.int32)
    for j in range(_E):
        pj = p[:, j:j + 1]
        rank = rank + (pj > p).astype(jnp.int32) \
                    + ((pj == p) & (j < col)).astype(jnp.int32)
    m0 = rank == 0
    m1 = rank == 1
    e0 = jnp.sum(jnp.where(m0, col, 0), axis=1)
    e1 = jnp.sum(jnp.where(m1, col, 0), axis=1)
    w0 = jnp.sum(jnp.where(m0, p, 0.0), axis=1)
    w1 = jnp.sum(jnp.where(m1, p, 0.0), axis=1)
    s = w0 + w1
    w_ref[0, :] = w0 / s
    w_ref[1, :] = w1 / s
    eid_ref[0, :] = e0
    eid_ref[1, :] = e1
    row = lax.broadcasted_iota(jnp.int32, (_NS // 2, _T), 0)
    tcol = lax.broadcasted_iota(jnp.int32, (_NS // 2, _T), 1)
    blockmask = jnp.where((tcol >> 8) == row, 1.0, 0.0)
    h0 = jnp.dot(blockmask, jnp.where(m0, 1.0, 0.0),
                 preferred_element_type=jnp.float32)
    h1 = jnp.dot(blockmask, jnp.where(m1, 1.0, 0.0),
                 preferred_element_type=jnp.float32)
    pad = jnp.zeros((_NS // 2, _L - _E), jnp.float32)
    hist = jnp.concatenate(
        [jnp.concatenate([h0, pad], axis=1),
         jnp.concatenate([h1, pad], axis=1)], axis=0)
    hist_ref[...] = hist.astype(jnp.int32)


def _router(x, gate_w, interpret=False):
    return pl.pallas_call(
        _router_body,
        out_shape=(
            jax.ShapeDtypeStruct((_K, _T), jnp.float32),
            jax.ShapeDtypeStruct((_K, _T), jnp.int32),
            jax.ShapeDtypeStruct((_NS, _L), jnp.int32),
        ),
        interpret=interpret,
    )(x, gate_w)


def _lane():
    return lax.iota(jnp.int32, _L)


def _permute(v, idx):
    return v.at[idx].get(mode="promise_in_bounds")


def _splat_sum(v):
    lane = _lane()
    for k in (1, 2, 4, 8):
        v = v + _permute(v, jnp.bitwise_xor(lane, k))
    return v


def _cumsum_incl(v):
    lane = _lane()
    onev = jnp.full((_L,), 1, jnp.int32)
    for lg in range(4):
        k = 1 << lg
        shifted = _permute(v, jnp.maximum(lane - k, 0))
        mk = jnp.minimum(lax.shift_right_logical(lane, lg), onev)
        v = v + shifted * mk
    return v


def _lane_splat(v, e):
    return _permute(v, jnp.full((_L,), e, jnp.int32))


def _dispatch_body(eids_hbm, hist_hbm, hid_hbm, xs_hbm, pos_hbm, te_hbm,
                   eids_v, hist_v, dest2_v, tev_v, rows_v, sem):
    c = lax.axis_index("c")
    s = lax.axis_index("s")
    lane = lax.iota(jnp.int32, _L)

    one = jnp.full((_L,), 1, jnp.int32)
    zero16 = jnp.full((_L,), 0, jnp.int32)

    pltpu.sync_copy(eids_hbm.at[pl.ds(s * 256, 256)], eids_v)
    pltpu.sync_copy(hist_hbm, hist_v)

    tot = jnp.zeros((_L,), jnp.int32)
    b = jnp.zeros((_L,), jnp.int32)
    for sp in range(_NS):
        row = hist_v[sp, :]
        tot = tot + row
        flag = jnp.where(sp < s, 1, 0)
        b = b + row * flag
    aligned = jnp.bitwise_and(tot + (_B - 1), -_B)
    a_excl = _cumsum_incl(aligned) - aligned
    base_v = a_excl + b
    base = [_lane_splat(base_v, e) for e in range(_E)]

    @pl.when(jnp.logical_and(c == 0, s == 0))
    def _():
        for t3 in range(_NTE // _L):
            jv = (t3 * _L) + lane
            te = jnp.zeros((_L,), jnp.int32)
            for e in range(1, _E):
                a_e = _lane_splat(a_excl, e)
                te = te + jnp.where(jv * _B >= a_e, one, zero16)
            tev_v[pl.ds(t3 * _L, _L)] = te
        pltpu.sync_copy(tev_v, te_hbm)

    for i in range(16):
        v = eids_v[pl.ds(i * _L, _L)]
        dest = jnp.zeros((_L,), jnp.int32)
        for e in range(_E):
            mi = jnp.where(v == e, one, zero16)
            cs = _cumsum_incl(mi)
            dest = dest + mi * (base[e] + cs - 1)
            base[e] = base[e] + _lane_splat(cs, _L - 1)
        half = i // 8
        ch = (i % 8) // 4
        kvec = i % 4
        @pl.when(c == half)
        def _():
            dest2_v[ch, pl.ds(kvec * _L, _L)] = dest

    for ch in range(2):
        r0 = s * 256 + c * 128 + ch * 64
        t0 = pl.multiple_of(jnp.bitwise_and(r0, _T - 1), 64)
        pltpu.sync_copy(hid_hbm.at[pl.ds(t0, 64)], rows_v)
        pltpu.async_copy(rows_v, xs_hbm.at[dest2_v.at[ch]], sem).wait()
        pltpu.sync_copy(dest2_v.at[ch], pos_hbm.at[pl.ds(pl.multiple_of(r0, 64), 64)])


def _dispatch(eids_flat, hist, hidden, interpret=False):
    mesh = plsc.VectorSubcoreMesh(core_axis_name="c", subcore_axis_name="s",
                                  num_cores=_NC, num_subcores=_NS)
    return pl.kernel(
        _dispatch_body,
        out_type=(
            jax.ShapeDtypeStruct((_NPAD, _D), jnp.float32),
            jax.ShapeDtypeStruct((_N,), jnp.int32),
            jax.ShapeDtypeStruct((_NTE,), jnp.int32),
        ),
        mesh=mesh,
        scratch_types=(
            pltpu.VMEM((256,), jnp.int32),
            pltpu.VMEM((_NS, _L), jnp.int32),
            pltpu.VMEM((2, 64), jnp.int32),
            pltpu.VMEM((_NTE,), jnp.int32),
            pltpu.VMEM((64, _D), jnp.float32),
            pltpu.SemaphoreType.DMA,
        ),
        interpret=interpret,
    )(eids_flat, hist, hidden)


def _gmlp_body(te_ref, xs_ref, wg_ref, wu_ref, wd_ref, o_ref):
    j = pl.program_id(0)
    te = te_ref[j]
    x = xs_ref[...]
    g = jnp.dot(x, wg_ref[te], preferred_element_type=jnp.float32)
    u = jnp.dot(x, wu_ref[te], preferred_element_type=jnp.float32)
    h = (g * jax.nn.sigmoid(g)) * u
    o_ref[...] = jnp.dot(h, wd_ref[te], preferred_element_type=jnp.float32)


def _gmlp(te, xs, w_gate, w_up, w_down, interpret=False):
    grid_spec = pltpu.PrefetchScalarGridSpec(
        num_scalar_prefetch=1,
        grid=(_NT,),
        in_specs=[
            pl.BlockSpec((_B, _D), lambda j, te: (j, 0)),
            pl.BlockSpec((_E, _D, _F), lambda j, te: (0, 0, 0)),
            pl.BlockSpec((_E, _D, _F), lambda j, te: (0, 0, 0)),
            pl.BlockSpec((_E, _F, _D), lambda j, te: (0, 0, 0)),
        ],
        out_specs=pl.BlockSpec((_B, _D), lambda j, te: (j, 0)),
    )
    return pl.pallas_call(
        _gmlp_body,
        grid_spec=grid_spec,
        out_shape=jax.ShapeDtypeStruct((_NPAD, _D), jnp.float32),
        interpret=interpret,
    )(te, xs, w_gate, w_up, w_down)


def _combine_body(ys_hbm, pos_hbm, w_hbm, out_hbm,
                  idx0_v, idx1_v, w0_v, w1_v, rows0_v, rows1_v, out_v,
                  sem0, sem1):
    c = lax.axis_index("c")
    s = lax.axis_index("s")
    lane = lax.iota(jnp.int32, _L)
    g = s * _NC + c
    for ch in range(2):
        tb = g * 64 + ch * 32
        pltpu.sync_copy(pos_hbm.at[pl.ds(tb, 32)], idx0_v)
        pltpu.sync_copy(pos_hbm.at[pl.ds(_T + tb, 32)], idx1_v)
        pltpu.sync_copy(w_hbm.at[pl.ds(tb, 32)], w0_v)
        pltpu.sync_copy(w_hbm.at[pl.ds(_T + tb, 32)], w1_v)
        d0 = pltpu.async_copy(ys_hbm.at[idx0_v], rows0_v, sem0)
        d1 = pltpu.async_copy(ys_hbm.at[idx1_v], rows1_v, sem1)
        d0.wait()
        d1.wait()

        def token_body(j, _):
            wv0 = w0_v[pl.ds((j >> 4) * _L, _L)]
            wv1 = w1_v[pl.ds((j >> 4) * _L, _L)]
            jm = jnp.bitwise_and(j, _L - 1)
            w0s = _lane_splat(wv0, jm)
            w1s = _lane_splat(wv1, jm)

            def q_body(qb, _):
                for k in range(8):
                    off = qb * 128 + k * _L
                    r0 = rows0_v[j, pl.ds(off, _L)]
                    r1 = rows1_v[j, pl.ds(off, _L)]
                    out_v[j, pl.ds(off, _L)] = w0s * r0 + w1s * r1
                return 0

            lax.fori_loop(0, _D // 128, q_body, 0)
            return 0

        lax.fori_loop(0, 32, token_body, 0)
        pltpu.sync_copy(out_v, out_hbm.at[pl.ds(tb, 32)])


def _combine(ys, pos, w_flat, interpret=False):
    mesh = plsc.VectorSubcoreMesh(core_axis_name="c", subcore_axis_name="s",
                                  num_cores=_NC, num_subcores=_NS)
    return pl.kernel(
        _combine_body,
        out_type=jax.ShapeDtypeStruct((_T, _D), jnp.float32),
        mesh=mesh,
        scratch_types=(
            pltpu.VMEM((32,), jnp.int32),
            pltpu.VMEM((32,), jnp.int32),
            pltpu.VMEM((32,), jnp.float32),
            pltpu.VMEM((32,), jnp.float32),
            pltpu.VMEM((32, _D), jnp.float32),
            pltpu.VMEM((32, _D), jnp.float32),
            pltpu.VMEM((32, _D), jnp.float32),
            pltpu.SemaphoreType.DMA,
            pltpu.SemaphoreType.DMA,
        ),
        interpret=interpret,
    )(ys, pos, w_flat)


def _moe(hidden_states, gate_w, w_gate, w_up, w_down, interpret=False):
    w2, eid2, hist = _router(hidden_states, gate_w, interpret=interpret)
    eflat = eid2.reshape(-1)
    wflat = w2.reshape(-1)
    xs, pos, te = _dispatch(eflat, hist, hidden_states, interpret=interpret)
    ys = _gmlp(te, xs, w_gate, w_up, w_down, interpret=interpret)
    return _combine(ys, pos, wflat, interpret=interpret)


def _moe_k2_probe(hidden_states, gate_w, w_gate, w_up, w_down):
    w2, eid2 = _router(hidden_states, gate_w)
    xs, pos, te = _dispatch(eid2.reshape(-1), hidden_states)
    return xs


def kernel(hidden_states, gate_w, w_gate, w_up, w_down):
    return _moe(hidden_states, gate_w, w_gate, w_up, w_down)


# speedup vs baseline: 1.0036x; 1.0036x over previous
import functools

import numpy as _NP

import jax
import jax.numpy as jnp
from jax import lax
from jax.experimental import pallas as pl
from jax.experimental.pallas import tpu as pltpu
from jax.experimental.pallas import tpu_sc as plsc

_T, _D, _E, _F, _K = 2048, 1024, 8, 512, 2
_N = _T * _K
_B = 128
_NT = 40
_NPAD = _NT * _B
_NTE = 48
_NC, _NS, _L = 2, 16, 16


def _router_body(x_ref, gw_ref, w_ref, eid_ref, hist_ref):
    x = x_ref[...]
    logits = jnp.dot(x, gw_ref[...], preferred_element_type=jnp.float32)
    p = jax.nn.softmax(logits, axis=-1)
    col = lax.broadcasted_iota(jnp.int32, (_T, _E), 1)
    rank = jnp.zeros((_T, _E), jnp.int32)
    for j in range(_E):
        pj = p[:, j:j + 1]
        rank = rank + (pj > p).astype(jnp.int32) \
                    + ((pj == p) & (j < col)).astype(jnp.int32)
    m0 = rank == 0
    m1 = rank == 1
    e0 = jnp.sum(jnp.where(m0, col, 0), axis=1)
    e1 = jnp.sum(jnp.where(m1, col, 0), axis=1)
    w0 = jnp.sum(jnp.where(m0, p, 0.0), axis=1)
    w1 = jnp.sum(jnp.where(m1, p, 0.0), axis=1)
    s = w0 + w1
    w_ref[0, :] = w0 / s
    w_ref[1, :] = w1 / s
    eid_ref[0, :] = e0
    eid_ref[1, :] = e1
    row = lax.broadcasted_iota(jnp.int32, (_NS // 2, _T), 0)
    tcol = lax.broadcasted_iota(jnp.int32, (_NS // 2, _T), 1)
    blockmask = jnp.where((tcol >> 8) == row, 1.0, 0.0)
    h0 = jnp.dot(blockmask, jnp.where(m0, 1.0, 0.0),
                 preferred_element_type=jnp.float32)
    h1 = jnp.dot(blockmask, jnp.where(m1, 1.0, 0.0),
                 preferred_element_type=jnp.float32)
    pad = jnp.zeros((_NS // 2, _L - _E), jnp.float32)
    hist = jnp.concatenate(
        [jnp.concatenate([h0, pad], axis=1),
         jnp.concatenate([h1, pad], axis=1)], axis=0)
    hist_ref[...] = hist.astype(jnp.int32)


def _router(x, gate_w, interpret=False):
    return pl.pallas_call(
        _router_body,
        out_shape=(
            jax.ShapeDtypeStruct((_K, _T), jnp.float32),
            jax.ShapeDtypeStruct((_K, _T), jnp.int32),
            jax.ShapeDtypeStruct((_NS, _L), jnp.int32),
        ),
        interpret=interpret,
    )(x, gate_w)


def _lane():
    return lax.iota(jnp.int32, _L)


def _permute(v, idx):
    return v.at[idx].get(mode="promise_in_bounds")


def _splat_sum(v):
    lane = _lane()
    for k in (1, 2, 4, 8):
        v = v + _permute(v, jnp.bitwise_xor(lane, k))
    return v


def _cumsum_incl(v):
    lane = _lane()
    onev = jnp.full((_L,), 1, jnp.int32)
    for lg in range(4):
        k = 1 << lg
        shifted = _permute(v, jnp.maximum(lane - k, 0))
        mk = jnp.minimum(lax.shift_right_logical(lane, lg), onev)
        v = v + shifted * mk
    return v


def _lane_splat(v, e):
    return _permute(v, jnp.full((_L,), e, jnp.int32))


def _dispatch_body(eids_hbm, hist_hbm, hid_hbm, xs_hbm, pos_hbm, te_hbm,
                   eids_v, hist_v, dest2_v, dest3_v, tev_v, rows_v, rows2_v,
                   sem, semc, semr0, semr1):
    c = lax.axis_index("c")
    s = lax.axis_index("s")
    lane = lax.iota(jnp.int32, _L)

    one = jnp.full((_L,), 1, jnp.int32)
    zero16 = jnp.full((_L,), 0, jnp.int32)

    pltpu.sync_copy(eids_hbm.at[pl.ds(s * 256, 256)], eids_v)
    pltpu.sync_copy(hist_hbm, hist_v)

    tot = jnp.zeros((_L,), jnp.int32)
    b = jnp.zeros((_L,), jnp.int32)
    for sp in range(_NS):
        row = hist_v[sp, :]
        tot = tot + row
        flag = jnp.where(sp < s, 1, 0)
        b = b + row * flag
    aligned = jnp.bitwise_and(tot + (_B - 1), -_B)
    a_excl = _cumsum_incl(aligned) - aligned
    base_v = a_excl + b
    base = [_lane_splat(base_v, e) for e in range(_E)]

    @pl.when(jnp.logical_and(c == 0, s == 0))
    def _():
        for t3 in range(_NTE // _L):
            jv = (t3 * _L) + lane
            te = jnp.zeros((_L,), jnp.int32)
            for e in range(1, _E):
                a_e = _lane_splat(a_excl, e)
                te = te + jnp.where(jv * _B >= a_e, one, zero16)
            tev_v[pl.ds(t3 * _L, _L)] = te
        pltpu.sync_copy(tev_v, te_hbm)

    rhalf = s * 256 + c * 128
    th = pl.multiple_of(jnp.bitwise_and(rhalf, _T - 1), 64)
    bufs = (rows_v, rows2_v)
    lsems = (semr0, semr1)
    ssems = (sem, semc)
    d_load = [pltpu.async_copy(hid_hbm.at[pl.ds(th + 32 * ch, 32)],
                               bufs[ch], lsems[ch]) for ch in range(2)]

    for i in range(16):
        v = eids_v[pl.ds(i * _L, _L)]
        dest = jnp.zeros((_L,), jnp.int32)
        for e in range(_E):
            mi = jnp.where(v == e, one, zero16)
            cs = _cumsum_incl(mi)
            dest = dest + mi * (base[e] + cs - 1)
            base[e] = base[e] + _lane_splat(cs, _L - 1)
        half = i // 8
        q = i % 8
        ch = q // 2
        kvec = q % 2
        @pl.when(c == half)
        def _():
            dest2_v[ch, pl.ds(kvec * _L, _L)] = dest
            dest3_v[pl.ds(q * _L, _L)] = dest

    pltpu.sync_copy(dest3_v, pos_hbm.at[pl.ds(pl.multiple_of(rhalf, 64), 128)])
    d_sc = [None, None]
    for ch in range(4):
        bi = ch % 2
        d_load[bi].wait()
        d_sc[bi] = pltpu.async_copy(bufs[bi], xs_hbm.at[dest2_v.at[ch]],
                                    ssems[bi])
        if ch + 2 < 4:
            d_sc[bi].wait()
            d_load[bi] = pltpu.async_copy(
                hid_hbm.at[pl.ds(th + 32 * (ch + 2), 32)], bufs[bi], lsems[bi])
    d_sc[0].wait()
    d_sc[1].wait()


def _dispatch(eids_flat, hist, hidden, interpret=False):
    mesh = plsc.VectorSubcoreMesh(core_axis_name="c", subcore_axis_name="s",
                                  num_cores=_NC, num_subcores=_NS)
    return pl.kernel(
        _dispatch_body,
        out_type=(
            jax.ShapeDtypeStruct((_NPAD, _D), jnp.float32),
            jax.ShapeDtypeStruct((_N,), jnp.int32),
            jax.ShapeDtypeStruct((_NTE,), jnp.int32),
        ),
        mesh=mesh,
        scratch_types=(
            pltpu.VMEM((256,), jnp.int32),
            pltpu.VMEM((_NS, _L), jnp.int32),
            pltpu.VMEM((4, 32), jnp.int32),
            pltpu.VMEM((128,), jnp.int32),
            pltpu.VMEM((_NTE,), jnp.int32),
            pltpu.VMEM((32, _D), jnp.float32),
            pltpu.VMEM((32, _D), jnp.float32),
            pltpu.SemaphoreType.DMA,
            pltpu.SemaphoreType.DMA,
            pltpu.SemaphoreType.DMA,
            pltpu.SemaphoreType.DMA,
        ),
        interpret=interpret,
    )(eids_flat, hist, hidden)


def _gmlp_body(te_ref, xs_ref, wg_ref, wu_ref, wd_ref, o_ref):
    j = pl.program_id(0)
    te = te_ref[j]
    x = xs_ref[...]
    g = jnp.dot(x, wg_ref[te], preferred_element_type=jnp.float32)
    u = jnp.dot(x, wu_ref[te], preferred_element_type=jnp.float32)
    h = (g * jax.nn.sigmoid(g)) * u
    o_ref[...] = jnp.dot(h, wd_ref[te], preferred_element_type=jnp.float32)


def _gmlp(te, xs, w_gate, w_up, w_down, interpret=False):
    grid_spec = pltpu.PrefetchScalarGridSpec(
        num_scalar_prefetch=1,
        grid=(_NT,),
        in_specs=[
            pl.BlockSpec((_B, _D), lambda j, te: (j, 0)),
            pl.BlockSpec((_E, _D, _F), lambda j, te: (0, 0, 0)),
            pl.BlockSpec((_E, _D, _F), lambda j, te: (0, 0, 0)),
            pl.BlockSpec((_E, _F, _D), lambda j, te: (0, 0, 0)),
        ],
        out_specs=pl.BlockSpec((_B, _D), lambda j, te: (j, 0)),
    )
    return pl.pallas_call(
        _gmlp_body,
        grid_spec=grid_spec,
        out_shape=jax.ShapeDtypeStruct((_NPAD, _D), jnp.float32),
        interpret=interpret,
    )(te, xs, w_gate, w_up, w_down)


def _combine_body(ys_hbm, pos_hbm, w_hbm, out_hbm,
                  idx0_v, idx1_v, w0_v, w1_v, rows0_v, rows1_v, out_v,
                  sem0, sem1):
    c = lax.axis_index("c")
    s = lax.axis_index("s")
    lane = lax.iota(jnp.int32, _L)
    g = s * _NC + c
    for ch in range(2):
        tb = g * 64 + ch * 32
        pltpu.sync_copy(pos_hbm.at[pl.ds(tb, 32)], idx0_v)
        pltpu.sync_copy(pos_hbm.at[pl.ds(_T + tb, 32)], idx1_v)
        pltpu.sync_copy(w_hbm.at[pl.ds(tb, 32)], w0_v)
        pltpu.sync_copy(w_hbm.at[pl.ds(_T + tb, 32)], w1_v)
        d0 = pltpu.async_copy(ys_hbm.at[idx0_v], rows0_v, sem0)
        d1 = pltpu.async_copy(ys_hbm.at[idx1_v], rows1_v, sem1)
        d0.wait()
        d1.wait()

        def token_body(j, _):
            wv0 = w0_v[pl.ds((j >> 4) * _L, _L)]
            wv1 = w1_v[pl.ds((j >> 4) * _L, _L)]
            jm = jnp.bitwise_and(j, _L - 1)
            w0s = _lane_splat(wv0, jm)
            w1s = _lane_splat(wv1, jm)

            def q_body(qb, _):
                for k in range(8):
                    off = qb * 128 + k * _L
                    r0 = rows0_v[j, pl.ds(off, _L)]
                    r1 = rows1_v[j, pl.ds(off, _L)]
                    out_v[j, pl.ds(off, _L)] = w0s * r0 + w1s * r1
                return 0

            lax.fori_loop(0, _D // 128, q_body, 0)
            return 0

        lax.fori_loop(0, 32, token_body, 0)
        pltpu.sync_copy(out_v, out_hbm.at[pl.ds(tb, 32)])


def _combine(ys, pos, w_flat, interpret=False):
    mesh = plsc.VectorSubcoreMesh(core_axis_name="c", subcore_axis_name="s",
                                  num_cores=_NC, num_subcores=_NS)
    return pl.kernel(
        _combine_body,
        out_type=jax.ShapeDtypeStruct((_T, _D), jnp.float32),
        mesh=mesh,
        scratch_types=(
            pltpu.VMEM((32,), jnp.int32),
            pltpu.VMEM((32,), jnp.int32),
            pltpu.VMEM((32,), jnp.float32),
            pltpu.VMEM((32,), jnp.float32),
            pltpu.VMEM((32, _D), jnp.float32),
            pltpu.VMEM((32, _D), jnp.float32),
            pltpu.VMEM((32, _D), jnp.float32),
            pltpu.SemaphoreType.DMA,
            pltpu.SemaphoreType.DMA,
        ),
        interpret=interpret,
    )(ys, pos, w_flat)


def _moe(hidden_states, gate_w, w_gate, w_up, w_down, interpret=False):
    w2, eid2, hist = _router(hidden_states, gate_w, interpret=interpret)
    eflat = eid2.reshape(-1)
    wflat = w2.reshape(-1)
    xs, pos, te = _dispatch(eflat, hist, hidden_states, interpret=interpret)
    ys = _gmlp(te, xs, w_gate, w_up, w_down, interpret=interpret)
    return _combine(ys, pos, wflat, interpret=interpret)


def _moe_k2_probe(hidden_states, gate_w, w_gate, w_up, w_down):
    w2, eid2 = _router(hidden_states, gate_w)
    xs, pos, te = _dispatch(eid2.reshape(-1), hidden_states)
    return xs


def kernel(hidden_states, gate_w, w_gate, w_up, w_down):
    return _moe(hidden_states, gate_w, w_gate, w_up, w_down)


# speedup vs baseline: 1.0669x; 1.0630x over previous
import functools

import numpy as _NP

import jax
import jax.numpy as jnp
from jax import lax
from jax.experimental import pallas as pl
from jax.experimental.pallas import tpu as pltpu
from jax.experimental.pallas import tpu_sc as plsc

_T, _D, _E, _F, _K = 2048, 1024, 8, 512, 2
_N = _T * _K
_B = 256
_NT = 23
_NPAD = _NT * _B
_NTE = 48
_NC, _NS, _L = 2, 16, 16


def _router_body(x_ref, gw_ref, w_ref, eid_ref, hist_ref):
    x = x_ref[...]
    logits = jnp.dot(x, gw_ref[...], preferred_element_type=jnp.float32)
    p = jax.nn.softmax(logits, axis=-1)
    col = lax.broadcasted_iota(jnp.int32, (_T, _E), 1)
    rank = jnp.zeros((_T, _E), jnp.int32)
    for j in range(_E):
        pj = p[:, j:j + 1]
        rank = rank + (pj > p).astype(jnp.int32) \
                    + ((pj == p) & (j < col)).astype(jnp.int32)
    m0 = rank == 0
    m1 = rank == 1
    e0 = jnp.sum(jnp.where(m0, col, 0), axis=1)
    e1 = jnp.sum(jnp.where(m1, col, 0), axis=1)
    w0 = jnp.sum(jnp.where(m0, p, 0.0), axis=1)
    w1 = jnp.sum(jnp.where(m1, p, 0.0), axis=1)
    s = w0 + w1
    w_ref[0, :] = w0 / s
    w_ref[1, :] = w1 / s
    eid_ref[0, :] = e0
    eid_ref[1, :] = e1
    row = lax.broadcasted_iota(jnp.int32, (_NS // 2, _T), 0)
    tcol = lax.broadcasted_iota(jnp.int32, (_NS // 2, _T), 1)
    blockmask = jnp.where((tcol >> 8) == row, 1.0, 0.0)
    h0 = jnp.dot(blockmask, jnp.where(m0, 1.0, 0.0),
                 preferred_element_type=jnp.float32)
    h1 = jnp.dot(blockmask, jnp.where(m1, 1.0, 0.0),
                 preferred_element_type=jnp.float32)
    pad = jnp.zeros((_NS // 2, _L - _E), jnp.float32)
    hist = jnp.concatenate(
        [jnp.concatenate([h0, pad], axis=1),
         jnp.concatenate([h1, pad], axis=1)], axis=0)
    hist_ref[...] = hist.astype(jnp.int32)


def _router(x, gate_w, interpret=False):
    return pl.pallas_call(
        _router_body,
        out_shape=(
            jax.ShapeDtypeStruct((_K, _T), jnp.float32),
            jax.ShapeDtypeStruct((_K, _T), jnp.int32),
            jax.ShapeDtypeStruct((_NS, _L), jnp.int32),
        ),
        interpret=interpret,
    )(x, gate_w)


def _lane():
    return lax.iota(jnp.int32, _L)


def _permute(v, idx):
    return v.at[idx].get(mode="promise_in_bounds")


def _splat_sum(v):
    lane = _lane()
    for k in (1, 2, 4, 8):
        v = v + _permute(v, jnp.bitwise_xor(lane, k))
    return v


def _cumsum_incl(v):
    lane = _lane()
    onev = jnp.full((_L,), 1, jnp.int32)
    for lg in range(4):
        k = 1 << lg
        shifted = _permute(v, jnp.maximum(lane - k, 0))
        mk = jnp.minimum(lax.shift_right_logical(lane, lg), onev)
        v = v + shifted * mk
    return v


def _lane_splat(v, e):
    return _permute(v, jnp.full((_L,), e, jnp.int32))


def _dispatch_body(eids_hbm, hist_hbm, hid_hbm, xs_hbm, pos_hbm, te_hbm,
                   eids_v, hist_v, dest2_v, dest3_v, tev_v, rows_v, rows2_v,
                   sem, semc, semr0, semr1):
    c = lax.axis_index("c")
    s = lax.axis_index("s")
    lane = lax.iota(jnp.int32, _L)

    one = jnp.full((_L,), 1, jnp.int32)
    zero16 = jnp.full((_L,), 0, jnp.int32)

    pltpu.sync_copy(eids_hbm.at[pl.ds(s * 256, 256)], eids_v)
    pltpu.sync_copy(hist_hbm, hist_v)

    tot = jnp.zeros((_L,), jnp.int32)
    b = jnp.zeros((_L,), jnp.int32)
    for sp in range(_NS):
        row = hist_v[sp, :]
        tot = tot + row
        flag = jnp.where(sp < s, 1, 0)
        b = b + row * flag
    aligned = jnp.bitwise_and(tot + (_B - 1), -_B)
    a_excl = _cumsum_incl(aligned) - aligned
    base_v = a_excl + b
    base = [_lane_splat(base_v, e) for e in range(_E)]

    @pl.when(jnp.logical_and(c == 0, s == 0))
    def _():
        for t3 in range(_NTE // _L):
            jv = (t3 * _L) + lane
            te = jnp.zeros((_L,), jnp.int32)
            for e in range(1, _E):
                a_e = _lane_splat(a_excl, e)
                te = te + jnp.where(jv * _B >= a_e, one, zero16)
            tev_v[pl.ds(t3 * _L, _L)] = te
        pltpu.sync_copy(tev_v, te_hbm)

    rhalf = s * 256 + c * 128
    th = pl.multiple_of(jnp.bitwise_and(rhalf, _T - 1), 64)
    bufs = (rows_v, rows2_v)
    lsems = (semr0, semr1)
    ssems = (sem, semc)
    d_load = [pltpu.async_copy(hid_hbm.at[pl.ds(th + 32 * ch, 32)],
                               bufs[ch], lsems[ch]) for ch in range(2)]

    for i in range(16):
        v = eids_v[pl.ds(i * _L, _L)]
        dest = jnp.zeros((_L,), jnp.int32)
        for e in range(_E):
            mi = jnp.where(v == e, one, zero16)
            cs = _cumsum_incl(mi)
            dest = dest + mi * (base[e] + cs - 1)
            base[e] = base[e] + _lane_splat(cs, _L - 1)
        half = i // 8
        q = i % 8
        ch = q // 2
        kvec = q % 2
        @pl.when(c == half)
        def _():
            dest2_v[ch, pl.ds(kvec * _L, _L)] = dest
            dest3_v[pl.ds(q * _L, _L)] = dest

    pltpu.sync_copy(dest3_v, pos_hbm.at[pl.ds(pl.multiple_of(rhalf, 64), 128)])
    d_sc = [None, None]
    for ch in range(4):
        bi = ch % 2
        d_load[bi].wait()
        d_sc[bi] = pltpu.async_copy(bufs[bi], xs_hbm.at[dest2_v.at[ch]],
                                    ssems[bi])
        if ch + 2 < 4:
            d_sc[bi].wait()
            d_load[bi] = pltpu.async_copy(
                hid_hbm.at[pl.ds(th + 32 * (ch + 2), 32)], bufs[bi], lsems[bi])
    d_sc[0].wait()
    d_sc[1].wait()


def _dispatch(eids_flat, hist, hidden, interpret=False):
    mesh = plsc.VectorSubcoreMesh(core_axis_name="c", subcore_axis_name="s",
                                  num_cores=_NC, num_subcores=_NS)
    return pl.kernel(
        _dispatch_body,
        out_type=(
            jax.ShapeDtypeStruct((_NPAD, _D), jnp.float32),
            jax.ShapeDtypeStruct((_N,), jnp.int32),
            jax.ShapeDtypeStruct((_NTE,), jnp.int32),
        ),
        mesh=mesh,
        scratch_types=(
            pltpu.VMEM((256,), jnp.int32),
            pltpu.VMEM((_NS, _L), jnp.int32),
            pltpu.VMEM((4, 32), jnp.int32),
            pltpu.VMEM((128,), jnp.int32),
            pltpu.VMEM((_NTE,), jnp.int32),
            pltpu.VMEM((32, _D), jnp.float32),
            pltpu.VMEM((32, _D), jnp.float32),
            pltpu.SemaphoreType.DMA,
            pltpu.SemaphoreType.DMA,
            pltpu.SemaphoreType.DMA,
            pltpu.SemaphoreType.DMA,
        ),
        interpret=interpret,
    )(eids_flat, hist, hidden)


def _gmlp_body(te_ref, xs_ref, wg_ref, wu_ref, wd_ref, o_ref):
    j = pl.program_id(0)
    te = te_ref[j]
    x = xs_ref[...]
    g = jnp.dot(x, wg_ref[te], preferred_element_type=jnp.float32)
    u = jnp.dot(x, wu_ref[te], preferred_element_type=jnp.float32)
    h = (g * jax.nn.sigmoid(g)) * u
    o_ref[...] = jnp.dot(h, wd_ref[te], preferred_element_type=jnp.float32)


def _gmlp(te, xs, w_gate, w_up, w_down, interpret=False):
    grid_spec = pltpu.PrefetchScalarGridSpec(
        num_scalar_prefetch=1,
        grid=(_NT,),
        in_specs=[
            pl.BlockSpec((_B, _D), lambda j, te: (j, 0)),
            pl.BlockSpec((_E, _D, _F), lambda j, te: (0, 0, 0)),
            pl.BlockSpec((_E, _D, _F), lambda j, te: (0, 0, 0)),
            pl.BlockSpec((_E, _F, _D), lambda j, te: (0, 0, 0)),
        ],
        out_specs=pl.BlockSpec((_B, _D), lambda j, te: (j, 0)),
    )
    return pl.pallas_call(
        _gmlp_body,
        grid_spec=grid_spec,
        out_shape=jax.ShapeDtypeStruct((_NPAD, _D), jnp.float32),
        interpret=interpret,
    )(te, xs, w_gate, w_up, w_down)


def _combine_body(ys_hbm, pos_hbm, w_hbm, out_hbm,
                  idx0_v, idx1_v, w0_v, w1_v, rows0_v, rows1_v, out_v,
                  sem0, sem1):
    c = lax.axis_index("c")
    s = lax.axis_index("s")
    lane = lax.iota(jnp.int32, _L)
    g = s * _NC + c
    for ch in range(2):
        tb = g * 64 + ch * 32
        pltpu.sync_copy(pos_hbm.at[pl.ds(tb, 32)], idx0_v)
        pltpu.sync_copy(pos_hbm.at[pl.ds(_T + tb, 32)], idx1_v)
        pltpu.sync_copy(w_hbm.at[pl.ds(tb, 32)], w0_v)
        pltpu.sync_copy(w_hbm.at[pl.ds(_T + tb, 32)], w1_v)
        d0 = pltpu.async_copy(ys_hbm.at[idx0_v], rows0_v, sem0)
        d1 = pltpu.async_copy(ys_hbm.at[idx1_v], rows1_v, sem1)
        d0.wait()
        d1.wait()

        def token_body(j, _):
            wv0 = w0_v[pl.ds((j >> 4) * _L, _L)]
            wv1 = w1_v[pl.ds((j >> 4) * _L, _L)]
            jm = jnp.bitwise_and(j, _L - 1)
            w0s = _lane_splat(wv0, jm)
            w1s = _lane_splat(wv1, jm)

            def q_body(qb, _):
                for k in range(8):
                    off = qb * 128 + k * _L
                    r0 = rows0_v[j, pl.ds(off, _L)]
                    r1 = rows1_v[j, pl.ds(off, _L)]
                    out_v[j, pl.ds(off, _L)] = w0s * r0 + w1s * r1
                return 0

            lax.fori_loop(0, _D // 128, q_body, 0)
            return 0

        lax.fori_loop(0, 32, token_body, 0)
        pltpu.sync_copy(out_v, out_hbm.at[pl.ds(tb, 32)])


def _combine(ys, pos, w_flat, interpret=False):
    mesh = plsc.VectorSubcoreMesh(core_axis_name="c", subcore_axis_name="s",
                                  num_cores=_NC, num_subcores=_NS)
    return pl.kernel(
        _combine_body,
        out_type=jax.ShapeDtypeStruct((_T, _D), jnp.float32),
        mesh=mesh,
        scratch_types=(
            pltpu.VMEM((32,), jnp.int32),
            pltpu.VMEM((32,), jnp.int32),
            pltpu.VMEM((32,), jnp.float32),
            pltpu.VMEM((32,), jnp.float32),
            pltpu.VMEM((32, _D), jnp.float32),
            pltpu.VMEM((32, _D), jnp.float32),
            pltpu.VMEM((32, _D), jnp.float32),
            pltpu.SemaphoreType.DMA,
            pltpu.SemaphoreType.DMA,
        ),
        interpret=interpret,
    )(ys, pos, w_flat)


def _moe(hidden_states, gate_w, w_gate, w_up, w_down, interpret=False):
    w2, eid2, hist = _router(hidden_states, gate_w, interpret=interpret)
    eflat = eid2.reshape(-1)
    wflat = w2.reshape(-1)
    xs, pos, te = _dispatch(eflat, hist, hidden_states, interpret=interpret)
    ys = _gmlp(te, xs, w_gate, w_up, w_down, interpret=interpret)
    return _combine(ys, pos, wflat, interpret=interpret)


def _moe_k2_probe(hidden_states, gate_w, w_gate, w_up, w_down):
    w2, eid2 = _router(hidden_states, gate_w)
    xs, pos, te = _dispatch(eid2.reshape(-1), hidden_states)
    return xs


def kernel(hidden_states, gate_w, w_gate, w_up, w_down):
    return _moe(hidden_states, gate_w, w_gate, w_up, w_down)


# speedup vs baseline: 1.0745x; 1.0071x over previous
import functools

import numpy as _NP

import jax
import jax.numpy as jnp
from jax import lax
from jax.experimental import pallas as pl
from jax.experimental.pallas import tpu as pltpu
from jax.experimental.pallas import tpu_sc as plsc

_T, _D, _E, _F, _K = 2048, 1024, 8, 512, 2
_N = _T * _K
_B = 256
_NT = 23
_NPAD = _NT * _B
_NTE = 48
_NC, _NS, _L = 2, 16, 16


_RB = 256


def _router_body(x_ref, gw_ref, w_ref, eid_ref, hist_ref):
    x = x_ref[...]
    logits = jnp.dot(x, gw_ref[...], preferred_element_type=jnp.float32)
    p = jax.nn.softmax(logits, axis=-1)
    col = lax.broadcasted_iota(jnp.int32, (_RB, _E), 1)
    rank = jnp.zeros((_RB, _E), jnp.int32)
    for j in range(_E):
        pj = p[:, j:j + 1]
        rank = rank + (pj > p).astype(jnp.int32) \
                    + ((pj == p) & (j < col)).astype(jnp.int32)
    m0 = rank == 0
    m1 = rank == 1
    e0 = jnp.sum(jnp.where(m0, col, 0), axis=1)
    e1 = jnp.sum(jnp.where(m1, col, 0), axis=1)
    w0 = jnp.sum(jnp.where(m0, p, 0.0), axis=1)
    w1 = jnp.sum(jnp.where(m1, p, 0.0), axis=1)
    s = w0 + w1
    w_ref[0, :] = w0 / s
    w_ref[1, :] = w1 / s
    eid_ref[0, :] = e0
    eid_ref[1, :] = e1
    ones = jnp.ones((1, _RB), jnp.float32)
    h0 = jnp.dot(ones, jnp.where(m0, 1.0, 0.0),
                 preferred_element_type=jnp.float32)
    h1 = jnp.dot(ones, jnp.where(m1, 1.0, 0.0),
                 preferred_element_type=jnp.float32)
    pad = jnp.zeros((1, _L - _E), jnp.float32)
    hist = jnp.concatenate(
        [jnp.concatenate([h0, pad], axis=1),
         jnp.concatenate([h1, pad], axis=1)], axis=0)
    hist_ref[0] = hist.astype(jnp.int32)


def _router(x, gate_w, interpret=False):
    nb = _T // _RB
    return pl.pallas_call(
        _router_body,
        grid=(nb,),
        in_specs=[
            pl.BlockSpec((_RB, _D), lambda i: (i, 0)),
            pl.BlockSpec((_D, _E), lambda i: (0, 0)),
        ],
        out_specs=(
            pl.BlockSpec((_K, _RB), lambda i: (0, i)),
            pl.BlockSpec((_K, _RB), lambda i: (0, i)),
            pl.BlockSpec((1, _K, _L), lambda i: (i, 0, 0)),
        ),
        out_shape=(
            jax.ShapeDtypeStruct((_K, _T), jnp.float32),
            jax.ShapeDtypeStruct((_K, _T), jnp.int32),
            jax.ShapeDtypeStruct((_T // _RB, _K, _L), jnp.int32),
        ),
        interpret=interpret,
    )(x, gate_w)


def _lane():
    return lax.iota(jnp.int32, _L)


def _permute(v, idx):
    return v.at[idx].get(mode="promise_in_bounds")


def _splat_sum(v):
    lane = _lane()
    for k in (1, 2, 4, 8):
        v = v + _permute(v, jnp.bitwise_xor(lane, k))
    return v


def _cumsum_incl(v):
    lane = _lane()
    onev = jnp.full((_L,), 1, jnp.int32)
    for lg in range(4):
        k = 1 << lg
        shifted = _permute(v, jnp.maximum(lane - k, 0))
        mk = jnp.minimum(lax.shift_right_logical(lane, lg), onev)
        v = v + shifted * mk
    return v


def _lane_splat(v, e):
    return _permute(v, jnp.full((_L,), e, jnp.int32))


def _dispatch_body(eids_hbm, hist_hbm, hid_hbm, xs_hbm, pos_hbm, te_hbm,
                   eids_v, hist_v, dest2_v, dest3_v, tev_v, rows_v, rows2_v,
                   sem, semc, semr0, semr1):
    c = lax.axis_index("c")
    s = lax.axis_index("s")
    lane = lax.iota(jnp.int32, _L)

    one = jnp.full((_L,), 1, jnp.int32)
    zero16 = jnp.full((_L,), 0, jnp.int32)

    pltpu.sync_copy(eids_hbm.at[pl.ds(s * 256, 256)], eids_v)
    pltpu.sync_copy(hist_hbm, hist_v)

    tot = jnp.zeros((_L,), jnp.int32)
    b = jnp.zeros((_L,), jnp.int32)
    for sp in range(_NS):
        row = hist_v[(sp % 8) * 2 + (sp // 8), :]
        tot = tot + row
        flag = jnp.where(sp < s, 1, 0)
        b = b + row * flag
    aligned = jnp.bitwise_and(tot + (_B - 1), -_B)
    a_excl = _cumsum_incl(aligned) - aligned
    base_v = a_excl + b
    base = [_lane_splat(base_v, e) for e in range(_E)]

    @pl.when(jnp.logical_and(c == 0, s == 0))
    def _():
        for t3 in range(_NTE // _L):
            jv = (t3 * _L) + lane
            te = jnp.zeros((_L,), jnp.int32)
            for e in range(1, _E):
                a_e = _lane_splat(a_excl, e)
                te = te + jnp.where(jv * _B >= a_e, one, zero16)
            tev_v[pl.ds(t3 * _L, _L)] = te
        pltpu.sync_copy(tev_v, te_hbm)

    rhalf = s * 256 + c * 128
    th = pl.multiple_of(jnp.bitwise_and(rhalf, _T - 1), 64)
    bufs = (rows_v, rows2_v)
    lsems = (semr0, semr1)
    ssems = (sem, semc)
    d_load = [pltpu.async_copy(hid_hbm.at[pl.ds(th + 32 * ch, 32)],
                               bufs[ch], lsems[ch]) for ch in range(2)]

    for i in range(16):
        v = eids_v[pl.ds(i * _L, _L)]
        dest = jnp.zeros((_L,), jnp.int32)
        for e in range(_E):
            mi = jnp.where(v == e, one, zero16)
            cs = _cumsum_incl(mi)
            dest = dest + mi * (base[e] + cs - 1)
            base[e] = base[e] + _lane_splat(cs, _L - 1)
        half = i // 8
        q = i % 8
        ch = q // 2
        kvec = q % 2
        @pl.when(c == half)
        def _():
            dest2_v[ch, pl.ds(kvec * _L, _L)] = dest
            dest3_v[pl.ds(q * _L, _L)] = dest

    pltpu.sync_copy(dest3_v, pos_hbm.at[pl.ds(pl.multiple_of(rhalf, 64), 128)])
    d_sc = [None, None]
    for ch in range(4):
        bi = ch % 2
        d_load[bi].wait()
        d_sc[bi] = pltpu.async_copy(bufs[bi], xs_hbm.at[dest2_v.at[ch]],
                                    ssems[bi])
        if ch + 2 < 4:
            d_sc[bi].wait()
            d_load[bi] = pltpu.async_copy(
                hid_hbm.at[pl.ds(th + 32 * (ch + 2), 32)], bufs[bi], lsems[bi])
    d_sc[0].wait()
    d_sc[1].wait()


def _dispatch(eids_flat, hist, hidden, interpret=False):
    mesh = plsc.VectorSubcoreMesh(core_axis_name="c", subcore_axis_name="s",
                                  num_cores=_NC, num_subcores=_NS)
    return pl.kernel(
        _dispatch_body,
        out_type=(
            jax.ShapeDtypeStruct((_NPAD, _D), jnp.float32),
            jax.ShapeDtypeStruct((_N,), jnp.int32),
            jax.ShapeDtypeStruct((_NTE,), jnp.int32),
        ),
        mesh=mesh,
        scratch_types=(
            pltpu.VMEM((256,), jnp.int32),
            pltpu.VMEM((_NS, _L), jnp.int32),
            pltpu.VMEM((4, 32), jnp.int32),
            pltpu.VMEM((128,), jnp.int32),
            pltpu.VMEM((_NTE,), jnp.int32),
            pltpu.VMEM((32, _D), jnp.float32),
            pltpu.VMEM((32, _D), jnp.float32),
            pltpu.SemaphoreType.DMA,
            pltpu.SemaphoreType.DMA,
            pltpu.SemaphoreType.DMA,
            pltpu.SemaphoreType.DMA,
        ),
        interpret=interpret,
    )(eids_flat, hist, hidden)


def _gmlp_body(te_ref, xs_ref, wg_ref, wu_ref, wd_ref, o_ref):
    j = pl.program_id(0)
    te = te_ref[j]
    x = xs_ref[...]
    g = jnp.dot(x, wg_ref[te], preferred_element_type=jnp.float32)
    u = jnp.dot(x, wu_ref[te], preferred_element_type=jnp.float32)
    h = (g * jax.nn.sigmoid(g)) * u
    o_ref[...] = jnp.dot(h, wd_ref[te], preferred_element_type=jnp.float32)


def _gmlp(te, xs, w_gate, w_up, w_down, interpret=False):
    grid_spec = pltpu.PrefetchScalarGridSpec(
        num_scalar_prefetch=1,
        grid=(_NT,),
        in_specs=[
            pl.BlockSpec((_B, _D), lambda j, te: (j, 0)),
            pl.BlockSpec((_E, _D, _F), lambda j, te: (0, 0, 0)),
            pl.BlockSpec((_E, _D, _F), lambda j, te: (0, 0, 0)),
            pl.BlockSpec((_E, _F, _D), lambda j, te: (0, 0, 0)),
        ],
        out_specs=pl.BlockSpec((_B, _D), lambda j, te: (j, 0)),
    )
    return pl.pallas_call(
        _gmlp_body,
        grid_spec=grid_spec,
        out_shape=jax.ShapeDtypeStruct((_NPAD, _D), jnp.float32),
        interpret=interpret,
    )(te, xs, w_gate, w_up, w_down)


def _combine_body(ys_hbm, pos_hbm, w_hbm, out_hbm,
                  idx0_v, idx1_v, w0_v, w1_v, rows0_v, rows1_v, out_v,
                  sem0, sem1):
    c = lax.axis_index("c")
    s = lax.axis_index("s")
    lane = lax.iota(jnp.int32, _L)
    g = s * _NC + c
    for ch in range(2):
        tb = g * 64 + ch * 32
        pltpu.sync_copy(pos_hbm.at[pl.ds(tb, 32)], idx0_v)
        pltpu.sync_copy(pos_hbm.at[pl.ds(_T + tb, 32)], idx1_v)
        pltpu.sync_copy(w_hbm.at[pl.ds(tb, 32)], w0_v)
        pltpu.sync_copy(w_hbm.at[pl.ds(_T + tb, 32)], w1_v)
        d0 = pltpu.async_copy(ys_hbm.at[idx0_v], rows0_v, sem0)
        d1 = pltpu.async_copy(ys_hbm.at[idx1_v], rows1_v, sem1)
        d0.wait()
        d1.wait()

        def token_body(j, _):
            wv0 = w0_v[pl.ds((j >> 4) * _L, _L)]
            wv1 = w1_v[pl.ds((j >> 4) * _L, _L)]
            jm = jnp.bitwise_and(j, _L - 1)
            w0s = _lane_splat(wv0, jm)
            w1s = _lane_splat(wv1, jm)

            def q_body(qb, _):
                for k in range(8):
                    off = qb * 128 + k * _L
                    r0 = rows0_v[j, pl.ds(off, _L)]
                    r1 = rows1_v[j, pl.ds(off, _L)]
                    out_v[j, pl.ds(off, _L)] = w0s * r0 + w1s * r1
                return 0

            lax.fori_loop(0, _D // 128, q_body, 0)
            return 0

        lax.fori_loop(0, 32, token_body, 0)
        pltpu.sync_copy(out_v, out_hbm.at[pl.ds(tb, 32)])


def _combine(ys, pos, w_flat, interpret=False):
    mesh = plsc.VectorSubcoreMesh(core_axis_name="c", subcore_axis_name="s",
                                  num_cores=_NC, num_subcores=_NS)
    return pl.kernel(
        _combine_body,
        out_type=jax.ShapeDtypeStruct((_T, _D), jnp.float32),
        mesh=mesh,
        scratch_types=(
            pltpu.VMEM((32,), jnp.int32),
            pltpu.VMEM((32,), jnp.int32),
            pltpu.VMEM((32,), jnp.float32),
            pltpu.VMEM((32,), jnp.float32),
            pltpu.VMEM((32, _D), jnp.float32),
            pltpu.VMEM((32, _D), jnp.float32),
            pltpu.VMEM((32, _D), jnp.float32),
            pltpu.SemaphoreType.DMA,
            pltpu.SemaphoreType.DMA,
        ),
        interpret=interpret,
    )(ys, pos, w_flat)


def _moe(hidden_states, gate_w, w_gate, w_up, w_down, interpret=False):
    w2, eid2, hist = _router(hidden_states, gate_w, interpret=interpret)
    eflat = eid2.reshape(-1)
    wflat = w2.reshape(-1)
    xs, pos, te = _dispatch(eflat, hist.reshape(_NS, _L), hidden_states,
                            interpret=interpret)
    ys = _gmlp(te, xs, w_gate, w_up, w_down, interpret=interpret)
    return _combine(ys, pos, wflat, interpret=interpret)


def _moe_k2_probe(hidden_states, gate_w, w_gate, w_up, w_down):
    w2, eid2 = _router(hidden_states, gate_w)
    xs, pos, te = _dispatch(eid2.reshape(-1), hidden_states)
    return xs


def kernel(hidden_states, gate_w, w_gate, w_up, w_down):
    return _moe(hidden_states, gate_w, w_gate, w_up, w_down)


# speedup vs baseline: 1.0881x; 1.0127x over previous
import functools

import numpy as _NP

import jax
import jax.numpy as jnp
from jax import lax
from jax.experimental import pallas as pl
from jax.experimental.pallas import tpu as pltpu
from jax.experimental.pallas import tpu_sc as plsc

_T, _D, _E, _F, _K = 2048, 1024, 8, 512, 2
_N = _T * _K
_B = 256
_NT = 23
_NPAD = _NT * _B
_NTE = 48
_NC, _NS, _L = 2, 16, 16


_RB = 256


def _router_body(x_ref, gw_ref, w_ref, eid_ref, hist_ref):
    x = x_ref[...]
    logits = jnp.dot(x, gw_ref[...], preferred_element_type=jnp.float32)
    p = jax.nn.softmax(logits, axis=-1)
    col = lax.broadcasted_iota(jnp.int32, (_RB, _E), 1)
    rank = jnp.zeros((_RB, _E), jnp.int32)
    for j in range(_E):
        pj = p[:, j:j + 1]
        rank = rank + (pj > p).astype(jnp.int32) \
                    + ((pj == p) & (j < col)).astype(jnp.int32)
    m0 = rank == 0
    m1 = rank == 1
    e0 = jnp.sum(jnp.where(m0, col, 0), axis=1)
    e1 = jnp.sum(jnp.where(m1, col, 0), axis=1)
    w0 = jnp.sum(jnp.where(m0, p, 0.0), axis=1)
    w1 = jnp.sum(jnp.where(m1, p, 0.0), axis=1)
    s = w0 + w1
    w_ref[0, :] = w0 / s
    w_ref[1, :] = w1 / s
    eid_ref[0, :] = e0
    eid_ref[1, :] = e1
    ones = jnp.ones((1, _RB), jnp.float32)
    h0 = jnp.dot(ones, jnp.where(m0, 1.0, 0.0),
                 preferred_element_type=jnp.float32)
    h1 = jnp.dot(ones, jnp.where(m1, 1.0, 0.0),
                 preferred_element_type=jnp.float32)
    pad = jnp.zeros((1, _L - _E), jnp.float32)
    hist = jnp.concatenate(
        [jnp.concatenate([h0, pad], axis=1),
         jnp.concatenate([h1, pad], axis=1)], axis=0)
    hist_ref[0] = hist.astype(jnp.int32)


def _router(x, gate_w, interpret=False):
    nb = _T // _RB
    return pl.pallas_call(
        _router_body,
        grid=(nb,),
        in_specs=[
            pl.BlockSpec((_RB, _D), lambda i: (i, 0)),
            pl.BlockSpec((_D, _E), lambda i: (0, 0)),
        ],
        out_specs=(
            pl.BlockSpec((_K, _RB), lambda i: (0, i)),
            pl.BlockSpec((_K, _RB), lambda i: (0, i)),
            pl.BlockSpec((1, _K, _L), lambda i: (i, 0, 0)),
        ),
        out_shape=(
            jax.ShapeDtypeStruct((_K, _T), jnp.float32),
            jax.ShapeDtypeStruct((_K, _T), jnp.int32),
            jax.ShapeDtypeStruct((_T // _RB, _K, _L), jnp.int32),
        ),
        interpret=interpret,
    )(x, gate_w)


def _lane():
    return lax.iota(jnp.int32, _L)


def _permute(v, idx):
    return v.at[idx].get(mode="promise_in_bounds")


def _splat_sum(v):
    lane = _lane()
    for k in (1, 2, 4, 8):
        v = v + _permute(v, jnp.bitwise_xor(lane, k))
    return v


def _cumsum_incl(v):
    lane = _lane()
    onev = jnp.full((_L,), 1, jnp.int32)
    for lg in range(4):
        k = 1 << lg
        shifted = _permute(v, jnp.maximum(lane - k, 0))
        mk = jnp.minimum(lax.shift_right_logical(lane, lg), onev)
        v = v + shifted * mk
    return v


def _lane_splat(v, e):
    return _permute(v, jnp.full((_L,), e, jnp.int32))


def _dispatch_body(eids_hbm, hist_hbm, hid_hbm, xs_hbm, pos_hbm, te_hbm,
                   eids_v, hist_v, dest2_v, dest3_v, tev_v, rows_v, rows2_v,
                   sem, semc, semr0, semr1):
    c = lax.axis_index("c")
    s = lax.axis_index("s")
    lane = lax.iota(jnp.int32, _L)

    one = jnp.full((_L,), 1, jnp.int32)
    zero16 = jnp.full((_L,), 0, jnp.int32)

    pltpu.sync_copy(eids_hbm.at[pl.ds(s * 256, 256)], eids_v)
    pltpu.sync_copy(hist_hbm, hist_v)

    tot = jnp.zeros((_L,), jnp.int32)
    b = jnp.zeros((_L,), jnp.int32)
    for sp in range(_NS):
        row = hist_v[(sp % 8) * 2 + (sp // 8), :]
        tot = tot + row
        flag = jnp.where(sp < s, 1, 0)
        b = b + row * flag
    aligned = jnp.bitwise_and(tot + (_B - 1), -_B)
    a_excl = _cumsum_incl(aligned) - aligned
    base_v = a_excl + b
    base = [_lane_splat(base_v, e) for e in range(_E)]

    @pl.when(jnp.logical_and(c == 0, s == 0))
    def _():
        ends = a_excl + tot
        for t3 in range(_NTE // _L):
            jv = (t3 * _L) + lane
            te = jnp.zeros((_L,), jnp.int32)
            for e in range(1, _E):
                a_e = _lane_splat(a_excl, e)
                te = te + jnp.where(jv * _B >= a_e, one, zero16)
            end_tile = _permute(ends, te)
            te = te + jnp.where(jv * _B >= end_tile, one, zero16) * _E
            tev_v[pl.ds(t3 * _L, _L)] = te
        pltpu.sync_copy(tev_v, te_hbm)

    rhalf = s * 256 + c * 128
    th = pl.multiple_of(jnp.bitwise_and(rhalf, _T - 1), 64)
    bufs = (rows_v, rows2_v)
    lsems = (semr0, semr1)
    ssems = (sem, semc)
    d_load = [pltpu.async_copy(hid_hbm.at[pl.ds(th + 32 * ch, 32)],
                               bufs[ch], lsems[ch]) for ch in range(2)]

    for i in range(16):
        v = eids_v[pl.ds(i * _L, _L)]
        dest = jnp.zeros((_L,), jnp.int32)
        for e in range(_E):
            mi = jnp.where(v == e, one, zero16)
            cs = _cumsum_incl(mi)
            dest = dest + mi * (base[e] + cs - 1)
            base[e] = base[e] + _lane_splat(cs, _L - 1)
        half = i // 8
        q = i % 8
        ch = q // 2
        kvec = q % 2
        @pl.when(c == half)
        def _():
            dest2_v[ch, pl.ds(kvec * _L, _L)] = dest
            dest3_v[pl.ds(q * _L, _L)] = dest

    pltpu.sync_copy(dest3_v, pos_hbm.at[pl.ds(pl.multiple_of(rhalf, 64), 128)])
    d_sc = [None, None]
    for ch in range(4):
        bi = ch % 2
        d_load[bi].wait()
        d_sc[bi] = pltpu.async_copy(bufs[bi], xs_hbm.at[dest2_v.at[ch]],
                                    ssems[bi])
        if ch + 2 < 4:
            d_sc[bi].wait()
            d_load[bi] = pltpu.async_copy(
                hid_hbm.at[pl.ds(th + 32 * (ch + 2), 32)], bufs[bi], lsems[bi])
    d_sc[0].wait()
    d_sc[1].wait()


def _dispatch(eids_flat, hist, hidden, interpret=False):
    mesh = plsc.VectorSubcoreMesh(core_axis_name="c", subcore_axis_name="s",
                                  num_cores=_NC, num_subcores=_NS)
    return pl.kernel(
        _dispatch_body,
        out_type=(
            jax.ShapeDtypeStruct((_NPAD, _D), jnp.float32),
            jax.ShapeDtypeStruct((_N,), jnp.int32),
            jax.ShapeDtypeStruct((_NTE,), jnp.int32),
        ),
        mesh=mesh,
        scratch_types=(
            pltpu.VMEM((256,), jnp.int32),
            pltpu.VMEM((_NS, _L), jnp.int32),
            pltpu.VMEM((4, 32), jnp.int32),
            pltpu.VMEM((128,), jnp.int32),
            pltpu.VMEM((_NTE,), jnp.int32),
            pltpu.VMEM((32, _D), jnp.float32),
            pltpu.VMEM((32, _D), jnp.float32),
            pltpu.SemaphoreType.DMA,
            pltpu.SemaphoreType.DMA,
            pltpu.SemaphoreType.DMA,
            pltpu.SemaphoreType.DMA,
        ),
        interpret=interpret,
    )(eids_flat, hist, hidden)


def _gmlp_body(te_ref, xs_ref, wg_ref, wu_ref, wd_ref, o_ref):
    j = pl.program_id(0)
    te = te_ref[j]

    @pl.when(te < _E)
    def _():
        x = xs_ref[...]
        g = jnp.dot(x, wg_ref[te], preferred_element_type=jnp.float32)
        u = jnp.dot(x, wu_ref[te], preferred_element_type=jnp.float32)
        h = (g * jax.nn.sigmoid(g)) * u
        o_ref[...] = jnp.dot(h, wd_ref[te], preferred_element_type=jnp.float32)


def _gmlp(te, xs, w_gate, w_up, w_down, interpret=False):
    grid_spec = pltpu.PrefetchScalarGridSpec(
        num_scalar_prefetch=1,
        grid=(_NT,),
        in_specs=[
            pl.BlockSpec((_B, _D), lambda j, te: (j, 0)),
            pl.BlockSpec((_E, _D, _F), lambda j, te: (0, 0, 0)),
            pl.BlockSpec((_E, _D, _F), lambda j, te: (0, 0, 0)),
            pl.BlockSpec((_E, _F, _D), lambda j, te: (0, 0, 0)),
        ],
        out_specs=pl.BlockSpec((_B, _D), lambda j, te: (j, 0)),
    )
    return pl.pallas_call(
        _gmlp_body,
        grid_spec=grid_spec,
        out_shape=jax.ShapeDtypeStruct((_NPAD, _D), jnp.float32),
        interpret=interpret,
    )(te, xs, w_gate, w_up, w_down)


def _combine_body(ys_hbm, pos_hbm, w_hbm, out_hbm,
                  idx0_v, idx1_v, w0_v, w1_v, rows0_v, rows1_v, out_v,
                  sem0, sem1):
    c = lax.axis_index("c")
    s = lax.axis_index("s")
    lane = lax.iota(jnp.int32, _L)
    g = s * _NC + c
    for ch in range(2):
        tb = g * 64 + ch * 32
        pltpu.sync_copy(pos_hbm.at[pl.ds(tb, 32)], idx0_v)
        pltpu.sync_copy(pos_hbm.at[pl.ds(_T + tb, 32)], idx1_v)
        pltpu.sync_copy(w_hbm.at[pl.ds(tb, 32)], w0_v)
        pltpu.sync_copy(w_hbm.at[pl.ds(_T + tb, 32)], w1_v)
        d0 = pltpu.async_copy(ys_hbm.at[idx0_v], rows0_v, sem0)
        d1 = pltpu.async_copy(ys_hbm.at[idx1_v], rows1_v, sem1)
        d0.wait()
        d1.wait()

        def token_body(j, _):
            wv0 = w0_v[pl.ds((j >> 4) * _L, _L)]
            wv1 = w1_v[pl.ds((j >> 4) * _L, _L)]
            jm = jnp.bitwise_and(j, _L - 1)
            w0s = _lane_splat(wv0, jm)
            w1s = _lane_splat(wv1, jm)

            def q_body(qb, _):
                for k in range(8):
                    off = qb * 128 + k * _L
                    r0 = rows0_v[j, pl.ds(off, _L)]
                    r1 = rows1_v[j, pl.ds(off, _L)]
                    out_v[j, pl.ds(off, _L)] = w0s * r0 + w1s * r1
                return 0

            lax.fori_loop(0, _D // 128, q_body, 0)
            return 0

        lax.fori_loop(0, 32, token_body, 0)
        pltpu.sync_copy(out_v, out_hbm.at[pl.ds(tb, 32)])


def _combine(ys, pos, w_flat, interpret=False):
    mesh = plsc.VectorSubcoreMesh(core_axis_name="c", subcore_axis_name="s",
                                  num_cores=_NC, num_subcores=_NS)
    return pl.kernel(
        _combine_body,
        out_type=jax.ShapeDtypeStruct((_T, _D), jnp.float32),
        mesh=mesh,
        scratch_types=(
            pltpu.VMEM((32,), jnp.int32),
            pltpu.VMEM((32,), jnp.int32),
            pltpu.VMEM((32,), jnp.float32),
            pltpu.VMEM((32,), jnp.float32),
            pltpu.VMEM((32, _D), jnp.float32),
            pltpu.VMEM((32, _D), jnp.float32),
            pltpu.VMEM((32, _D), jnp.float32),
            pltpu.SemaphoreType.DMA,
            pltpu.SemaphoreType.DMA,
        ),
        interpret=interpret,
    )(ys, pos, w_flat)


def _moe(hidden_states, gate_w, w_gate, w_up, w_down, interpret=False):
    w2, eid2, hist = _router(hidden_states, gate_w, interpret=interpret)
    eflat = eid2.reshape(-1)
    wflat = w2.reshape(-1)
    xs, pos, te = _dispatch(eflat, hist.reshape(_NS, _L), hidden_states,
                            interpret=interpret)
    ys = _gmlp(te, xs, w_gate, w_up, w_down, interpret=interpret)
    return _combine(ys, pos, wflat, interpret=interpret)


def _moe_k2_probe(hidden_states, gate_w, w_gate, w_up, w_down):
    w2, eid2 = _router(hidden_states, gate_w)
    xs, pos, te = _dispatch(eid2.reshape(-1), hidden_states)
    return xs


def kernel(hidden_states, gate_w, w_gate, w_up, w_down):
    return _moe(hidden_states, gate_w, w_gate, w_up, w_down)
